# baseline (speedup 1.0000x reference)
import functools

import jax
import jax.numpy as jnp
from jax.experimental import pallas as pl
from jax.experimental.pallas import tpu as pltpu
from jax.experimental.pallas import tpu_sc as plsc

_D = 128

_mesh = plsc.ScalarSubcoreMesh(axis_name="c", num_cores=1)


@functools.partial(
    pl.kernel,
    mesh=_mesh,
    out_type=jax.ShapeDtypeStruct((1, _D), jnp.float32),
    scratch_types=[
        pltpu.SMEM((1,), jnp.int32),
    ],
)
def _gather_row(x_hbm, emb_hbm, out_hbm, idx_s):
    pltpu.sync_copy(x_hbm, idx_s)
    i = idx_s[0]
    pltpu.sync_copy(emb_hbm.at[pl.ds(i, 1)], out_hbm)


def kernel(x, emb):
    return _gather_row(x.astype(jnp.int32), emb).reshape(64, 2)
